# baseline (speedup 1.0000x reference)
import functools

import jax
import jax.numpy as jnp
from jax import lax
from jax.experimental import pallas as pl
from jax.experimental.pallas import tpu as pltpu
from jax.experimental.pallas import tpu_sc as plsc

N = 10000
E = 320000
D = 128
DE = 16
ON = 128
DG = 64
OG = 64
A = 16

NC = 2
F = ON // NC
NS = 16
CH = 128
TCH = E // CH
BCH = TCH // NS
XCH = TCH - NS * BCH
RPT = 624
TAIL = N - NS * RPT

LANES = 16


def _xm_body(x_ref, w_ref, o_ref):
    o_ref[0] = jnp.dot(x_ref[...], w_ref[0],
                       preferred_element_type=jnp.float32
                       ).astype(jnp.bfloat16)


def _em_body(ef_ref, w_ref, b_ref, o_ref):
    o_ref[...] = (jnp.dot(ef_ref[...], w_ref[...],
                          preferred_element_type=jnp.float32)
                  + b_ref[...]).astype(jnp.bfloat16)


def _sc_body(xm_hbm, em_hbm, src_hbm, dst_hbm, zero_hbm, out_hbm,
             src_v, dst_v, rows0, rows1, rows2, msg0, msg1, msg2,
             sb0, sb1, sb2, acc_sh,
             gsem0, gsem1, gsem2, esem0, esem1, esem2,
             ssem0, ssem1, ssem2):
    cid = lax.axis_index("c")
    sid = lax.axis_index("s")
    row0 = sid * RPT
    cbase = sid * BCH
    coff = cid * F
    xm_c = xm_hbm.at[cid]
    slots = ((rows0, msg0, sb0, gsem0, esem0, ssem0),
             (rows1, msg1, sb1, gsem1, esem1, ssem1),
             (rows2, msg2, sb2, gsem2, esem2, ssem2))

    pltpu.sync_copy(src_hbm.at[pl.ds(cbase, BCH)], src_v.at[pl.ds(0, BCH)])
    pltpu.sync_copy(dst_hbm.at[pl.ds(cbase, BCH)], dst_v.at[pl.ds(0, BCH)])

    @pl.when(sid < XCH)
    def _stage_extra():
        pltpu.sync_copy(src_hbm.at[pl.ds(NS * BCH + sid, 1)],
                        src_v.at[pl.ds(BCH, 1)])
        pltpu.sync_copy(dst_hbm.at[pl.ds(NS * BCH + sid, 1)],
                        dst_v.at[pl.ds(BCH, 1)])

    def issue(j, cg, b):
        rows_b, msg_b, _, gsem_b, esem_b, _ = slots[b]
        pltpu.async_copy(xm_c.at[src_v.at[j]], rows_b, gsem_b)
        pltpu.async_copy(
            em_hbm.at[pl.ds(cg * CH, CH), pl.ds(coff, F)], msg_b, esem_b)

    def wait_loads(j, cg, b):
        rows_b, msg_b, _, gsem_b, esem_b, _ = slots[b]
        pltpu.make_async_copy(xm_c.at[src_v.at[j]], rows_b, gsem_b).wait()
        pltpu.make_async_copy(
            em_hbm.at[pl.ds(cg * CH, CH), pl.ds(coff, F)], msg_b,
            esem_b).wait()

    def compute(b):
        rows_b, msg_b, sb_b, _, _, _ = slots[b]

        def quad(q, c2):
            r = q * 4
            for dr in range(4):
                for c in range(F // 32):
                    s32 = pl.ds(c * 32, 32)
                    ra, rb = plsc.unpack(
                        rows_b[r + dr, s32],
                        format=plsc.PackFormat.INTERLEAVED,
                        preferred_element_type=jnp.float32)
                    ma, mb = plsc.unpack(
                        msg_b[r + dr, s32],
                        format=plsc.PackFormat.INTERLEAVED,
                        preferred_element_type=jnp.float32)
                    sb_b[r + dr, pl.ds(c * 32, LANES)] = jnp.maximum(
                        ra + ma, 0.0)
                    sb_b[r + dr, pl.ds(c * 32 + LANES, LANES)] = (
                        jnp.maximum(rb + mb, 0.0))
            return c2

        lax.fori_loop(0, CH // 4, quad, 0)

    def scatter(j, b):
        _, _, sb_b, _, _, ssem_b = slots[b]
        pltpu.async_copy(sb_b, acc_sh.at[dst_v.at[j]], ssem_b, add=True)

    def wait_scatter(j, b):
        _, _, sb_b, _, _, ssem_b = slots[b]
        pltpu.make_async_copy(sb_b, acc_sh.at[dst_v.at[j]], ssem_b).wait()

    pltpu.sync_copy(zero_hbm, acc_sh.at[pl.ds(row0, RPT)])

    @pl.when(sid == NS - 1)
    def _zero_tail():
        pltpu.sync_copy(zero_hbm.at[pl.ds(0, TAIL)],
                        acc_sh.at[pl.ds(N - TAIL, TAIL)])

    plsc.subcore_barrier()

    issue(0, cbase + 0, 0)
    issue(1, cbase + 1, 1)
    issue(2, cbase + 2, 2)

    def trip(it, carry):
        j0 = it * 3
        for b in range(3):
            j = j0 + b
            wait_loads(j, cbase + j, b)

            @pl.when(it > 0)
            def _drain():
                wait_scatter(j - 3, b)

            compute(b)
            scatter(j, b)

            @pl.when(j + 3 < BCH)
            def _next():
                issue(j + 3, cbase + j + 3, b)
        return carry

    lax.fori_loop(0, BCH // 3, trip, 0)
    wait_scatter(BCH - 3, 0)
    wait_scatter(BCH - 2, 1)
    wait_scatter(BCH - 1, 2)

    @pl.when(sid < XCH)
    def _extra():
        issue(BCH, NS * BCH + sid, 0)
        wait_loads(BCH, NS * BCH + sid, 0)
        compute(0)
        scatter(BCH, 0)
        wait_scatter(BCH, 0)

    plsc.subcore_barrier()
    pltpu.sync_copy(acc_sh.at[pl.ds(row0, RPT)],
                    out_hbm.at[cid, pl.ds(row0, RPT)])

    @pl.when(sid == NS - 1)
    def _write_tail():
        pltpu.sync_copy(acc_sh.at[pl.ds(N - TAIL, TAIL)],
                        out_hbm.at[cid, pl.ds(N - TAIL, TAIL)])


def _final_body(x_ref, p_ref, wn1_ref, wn2a_ref, wn2b_ref, bn_ref, gf_ref,
                wg_ref, bg_ref, wl1_ref, wl2_ref, bl_ref, am_ref, o_ref,
                acc):
    i = pl.program_id(0)

    @pl.when(i == 0)
    def _init():
        acc[...] = jnp.zeros_like(acc)

    h = jnp.maximum(
        jnp.dot(x_ref[...], wn1_ref[...], preferred_element_type=jnp.float32)
        + jnp.dot(p_ref[0], wn2a_ref[...], preferred_element_type=jnp.float32)
        + jnp.dot(p_ref[1], wn2b_ref[...], preferred_element_type=jnp.float32)
        + bn_ref[...], 0.0)
    acc[...] += jnp.sum(h, axis=0, keepdims=True)

    @pl.when(i == pl.num_programs(0) - 1)
    def _fin():
        emb_nodes = acc[...] / N
        emb_graph = jnp.dot(gf_ref[...], wg_ref[...],
                            preferred_element_type=jnp.float32) + bg_ref[...]
        logits = (jnp.dot(emb_nodes, wl1_ref[...],
                          preferred_element_type=jnp.float32)
                  + jnp.dot(emb_graph, wl2_ref[...],
                            preferred_element_type=jnp.float32)
                  + bl_ref[...])
        inf_mask = jnp.maximum(jnp.log(am_ref[...]),
                               jnp.finfo(jnp.float32).min)
        o_ref[...] = logits + inf_mask


_xm_call = pl.pallas_call(
    _xm_body,
    grid=(NC,),
    in_specs=[pl.BlockSpec((N, D), lambda j: (0, 0)),
              pl.BlockSpec((1, D, F), lambda j: (j, 0, 0))],
    out_specs=pl.BlockSpec((1, N, F), lambda j: (j, 0, 0)),
    out_shape=jax.ShapeDtypeStruct((NC, N, F), jnp.bfloat16),
)

_em_call = pl.pallas_call(
    _em_body,
    grid=(40,),
    in_specs=[pl.BlockSpec((E // 40, DE), lambda i: (i, 0)),
              pl.BlockSpec((DE, ON), lambda i: (0, 0)),
              pl.BlockSpec((1, ON), lambda i: (0, 0))],
    out_specs=pl.BlockSpec((E // 40, ON), lambda i: (i, 0)),
    out_shape=jax.ShapeDtypeStruct((E, ON), jnp.bfloat16),
)

_sc_call = functools.partial(
    pl.kernel,
    out_type=jax.ShapeDtypeStruct((NC, N, F), jnp.float32),
    mesh=plsc.VectorSubcoreMesh(core_axis_name="c", subcore_axis_name="s"),
    compiler_params=pltpu.CompilerParams(use_tc_tiling_on_sc=False,
                                         needs_layout_passes=False),
    scratch_types=[
        pltpu.VMEM((BCH + 1, CH), jnp.int32),
        pltpu.VMEM((BCH + 1, CH), jnp.int32),
        pltpu.VMEM((CH, F), jnp.bfloat16),
        pltpu.VMEM((CH, F), jnp.bfloat16),
        pltpu.VMEM((CH, F), jnp.bfloat16),
        pltpu.VMEM((CH, F), jnp.bfloat16),
        pltpu.VMEM((CH, F), jnp.bfloat16),
        pltpu.VMEM((CH, F), jnp.bfloat16),
        pltpu.VMEM((CH, F), jnp.float32),
        pltpu.VMEM((CH, F), jnp.float32),
        pltpu.VMEM((CH, F), jnp.float32),
        pltpu.VMEM_SHARED((N, F), jnp.float32),
        pltpu.SemaphoreType.DMA,
        pltpu.SemaphoreType.DMA,
        pltpu.SemaphoreType.DMA,
        pltpu.SemaphoreType.DMA,
        pltpu.SemaphoreType.DMA,
        pltpu.SemaphoreType.DMA,
        pltpu.SemaphoreType.DMA,
        pltpu.SemaphoreType.DMA,
        pltpu.SemaphoreType.DMA,
    ],
)(_sc_body)

_final_call = pl.pallas_call(
    _final_body,
    grid=(10,),
    in_specs=[pl.BlockSpec((N // 10, D), lambda i: (i, 0)),
              pl.BlockSpec((NC, N // 10, F), lambda i: (0, i, 0)),
              pl.BlockSpec((D, ON), lambda i: (0, 0)),
              pl.BlockSpec((F, ON), lambda i: (0, 0)),
              pl.BlockSpec((F, ON), lambda i: (0, 0)),
              pl.BlockSpec((1, ON), lambda i: (0, 0)),
              pl.BlockSpec((1, DG + A), lambda i: (0, 0)),
              pl.BlockSpec((DG + A, OG), lambda i: (0, 0)),
              pl.BlockSpec((1, OG), lambda i: (0, 0)),
              pl.BlockSpec((ON, A), lambda i: (0, 0)),
              pl.BlockSpec((OG, A), lambda i: (0, 0)),
              pl.BlockSpec((1, A), lambda i: (0, 0)),
              pl.BlockSpec((1, A), lambda i: (0, 0))],
    out_specs=pl.BlockSpec((1, A), lambda i: (0, 0)),
    out_shape=jax.ShapeDtypeStruct((1, A), jnp.float32),
    scratch_shapes=[pltpu.VMEM((1, ON), jnp.float32)],
)


def kernel(node_features, edge_features, edges_src, edges_dst,
           graph_features, action_mask,
           W_msg, b_msg, W_node, b_node, W_graph, b_graph,
           W_logit, b_logit):
    x = node_features[0]
    ef = edge_features[0]
    src3 = edges_src[0].reshape(TCH, CH)
    dst3 = edges_dst[0].reshape(TCH, CH)

    w1p = W_msg[:D].reshape(D, NC, F).transpose(1, 0, 2)

    xm = _xm_call(x, w1p)
    em = _em_call(ef, W_msg[D:], b_msg.reshape(1, ON))

    zeros = jnp.zeros((RPT, F), dtype=jnp.float32)
    halves = _sc_call(xm, em, src3, dst3, zeros)

    perm = jnp.concatenate(
        [jnp.concatenate([jnp.arange(c * 32, c * 32 + 32, 2),
                          jnp.arange(c * 32 + 1, c * 32 + 32, 2)])
         for c in range(F // 32)])
    logits = _final_call(
        x, halves, W_node[:D], W_node[D:D + F][perm],
        W_node[D + F:][perm],
        b_node.reshape(1, ON),
        graph_features, W_graph, b_graph.reshape(1, OG),
        W_logit[:ON], W_logit[ON:], b_logit.reshape(1, A), action_mask)
    return logits


# speedup vs baseline: 1.9813x; 1.9813x over previous
import functools

import jax
import jax.numpy as jnp
from jax import lax
from jax.experimental import pallas as pl
from jax.experimental.pallas import tpu as pltpu
from jax.experimental.pallas import tpu_sc as plsc

N = 10000
E = 320000
D = 128
DE = 16
ON = 128
DG = 64
OG = 64
A = 16

NC = 2
F = ON // NC
NS = 16
CH = 128
TCH = E // CH
BCH = TCH // NS
XCH = TCH - NS * BCH
RPT = 624
TAIL = N - NS * RPT

LANES = 16


def _xm_body(x_ref, w_ref, o_ref):
    o_ref[0] = jnp.dot(x_ref[...], w_ref[0],
                       preferred_element_type=jnp.float32)


def _em_body(ef_ref, w_ref, b_ref, o_ref):
    o_ref[...] = jnp.dot(ef_ref[...], w_ref[...],
                         preferred_element_type=jnp.float32) + b_ref[...]


def _sc_body(xm_hbm, em_hbm, src_hbm, dst_hbm, zero_hbm, out_hbm,
             src_v, dst_v, rows0, rows1, rows2, msg0, msg1, msg2, acc_sh,
             gsem0, gsem1, gsem2, esem0, esem1, esem2,
             ssem0, ssem1, ssem2):
    cid = lax.axis_index("c")
    sid = lax.axis_index("s")
    row0 = sid * RPT
    cbase = sid * BCH
    coff = cid * F
    xm_c = xm_hbm.at[cid]
    slots = ((rows0, msg0, None, gsem0, esem0, ssem0),
             (rows1, msg1, None, gsem1, esem1, ssem1),
             (rows2, msg2, None, gsem2, esem2, ssem2))

    pltpu.sync_copy(src_hbm.at[pl.ds(cbase, BCH)], src_v.at[pl.ds(0, BCH)])
    pltpu.sync_copy(dst_hbm.at[pl.ds(cbase, BCH)], dst_v.at[pl.ds(0, BCH)])

    @pl.when(sid < XCH)
    def _stage_extra():
        pltpu.sync_copy(src_hbm.at[pl.ds(NS * BCH + sid, 1)],
                        src_v.at[pl.ds(BCH, 1)])
        pltpu.sync_copy(dst_hbm.at[pl.ds(NS * BCH + sid, 1)],
                        dst_v.at[pl.ds(BCH, 1)])

    def issue(j, cg, b):
        rows_b, msg_b, _, gsem_b, esem_b, _ = slots[b]
        pltpu.async_copy(xm_c.at[src_v.at[j]], rows_b, gsem_b)
        pltpu.async_copy(
            em_hbm.at[pl.ds(cg * CH, CH), pl.ds(coff, F)], msg_b, esem_b)

    def wait_loads(j, cg, b):
        rows_b, msg_b, _, gsem_b, esem_b, _ = slots[b]
        pltpu.make_async_copy(xm_c.at[src_v.at[j]], rows_b, gsem_b).wait()
        pltpu.make_async_copy(
            em_hbm.at[pl.ds(cg * CH, CH), pl.ds(coff, F)], msg_b,
            esem_b).wait()

    def compute(b):
        rows_b, msg_b, _, _, _, _ = slots[b]

        def quad(q, c2):
            r = q * 4
            for dr in range(4):
                for c in range(F // LANES):
                    so = pl.ds(c * LANES, LANES)
                    rows_b[r + dr, so] = jnp.maximum(
                        rows_b[r + dr, so] + msg_b[r + dr, so], 0.0)
            return c2

        lax.fori_loop(0, CH // 4, quad, 0)

    def scatter(j, b):
        rows_b, _, _, _, _, ssem_b = slots[b]
        pltpu.async_copy(rows_b, acc_sh.at[dst_v.at[j]], ssem_b, add=True)

    def wait_scatter(j, b):
        rows_b, _, _, _, _, ssem_b = slots[b]
        pltpu.make_async_copy(rows_b, acc_sh.at[dst_v.at[j]],
                              ssem_b).wait()

    pltpu.sync_copy(zero_hbm, acc_sh.at[pl.ds(row0, RPT)])

    @pl.when(sid == NS - 1)
    def _zero_tail():
        pltpu.sync_copy(zero_hbm.at[pl.ds(0, TAIL)],
                        acc_sh.at[pl.ds(N - TAIL, TAIL)])

    plsc.subcore_barrier()

    issue(0, cbase + 0, 0)
    issue(1, cbase + 1, 1)
    issue(2, cbase + 2, 2)

    def trip(it, carry):
        j0 = it * 3
        for b in range(3):
            j = j0 + b
            wait_loads(j, cbase + j, b)

            @pl.when(it > 0)
            def _drain():
                wait_scatter(j - 3, b)

            compute(b)
            scatter(j, b)

            @pl.when(j + 3 < BCH)
            def _next():
                issue(j + 3, cbase + j + 3, b)
        return carry

    lax.fori_loop(0, BCH // 3, trip, 0)
    wait_scatter(BCH - 3, 0)
    wait_scatter(BCH - 2, 1)
    wait_scatter(BCH - 1, 2)

    @pl.when(sid < XCH)
    def _extra():
        issue(BCH, NS * BCH + sid, 0)
        wait_loads(BCH, NS * BCH + sid, 0)
        compute(0)
        scatter(BCH, 0)
        wait_scatter(BCH, 0)

    plsc.subcore_barrier()
    pltpu.sync_copy(acc_sh.at[pl.ds(row0, RPT)],
                    out_hbm.at[cid, pl.ds(row0, RPT)])

    @pl.when(sid == NS - 1)
    def _write_tail():
        pltpu.sync_copy(acc_sh.at[pl.ds(N - TAIL, TAIL)],
                        out_hbm.at[cid, pl.ds(N - TAIL, TAIL)])


def _final_body(x_ref, p_ref, wn1_ref, wn2a_ref, wn2b_ref, bn_ref, gf_ref,
                wg_ref, bg_ref, wl1_ref, wl2_ref, bl_ref, am_ref, o_ref,
                acc):
    i = pl.program_id(0)

    @pl.when(i == 0)
    def _init():
        acc[...] = jnp.zeros_like(acc)

    h = jnp.maximum(
        jnp.dot(x_ref[...], wn1_ref[...], preferred_element_type=jnp.float32)
        + jnp.dot(p_ref[0], wn2a_ref[...], preferred_element_type=jnp.float32)
        + jnp.dot(p_ref[1], wn2b_ref[...], preferred_element_type=jnp.float32)
        + bn_ref[...], 0.0)
    acc[...] += jnp.sum(h, axis=0, keepdims=True)

    @pl.when(i == pl.num_programs(0) - 1)
    def _fin():
        emb_nodes = acc[...] / N
        emb_graph = jnp.dot(gf_ref[...], wg_ref[...],
                            preferred_element_type=jnp.float32) + bg_ref[...]
        logits = (jnp.dot(emb_nodes, wl1_ref[...],
                          preferred_element_type=jnp.float32)
                  + jnp.dot(emb_graph, wl2_ref[...],
                            preferred_element_type=jnp.float32)
                  + bl_ref[...])
        inf_mask = jnp.maximum(jnp.log(am_ref[...]),
                               jnp.finfo(jnp.float32).min)
        o_ref[...] = logits + inf_mask


_xm_call = pl.pallas_call(
    _xm_body,
    grid=(NC,),
    in_specs=[pl.BlockSpec((N, D), lambda j: (0, 0)),
              pl.BlockSpec((1, D, F), lambda j: (j, 0, 0))],
    out_specs=pl.BlockSpec((1, N, F), lambda j: (j, 0, 0)),
    out_shape=jax.ShapeDtypeStruct((NC, N, F), jnp.float32),
)

_em_call = pl.pallas_call(
    _em_body,
    grid=(40,),
    in_specs=[pl.BlockSpec((E // 40, DE), lambda i: (i, 0)),
              pl.BlockSpec((DE, ON), lambda i: (0, 0)),
              pl.BlockSpec((1, ON), lambda i: (0, 0))],
    out_specs=pl.BlockSpec((E // 40, ON), lambda i: (i, 0)),
    out_shape=jax.ShapeDtypeStruct((E, ON), jnp.float32),
)

_sc_call = functools.partial(
    pl.kernel,
    out_type=jax.ShapeDtypeStruct((NC, N, F), jnp.float32),
    mesh=plsc.VectorSubcoreMesh(core_axis_name="c", subcore_axis_name="s"),
    compiler_params=pltpu.CompilerParams(use_tc_tiling_on_sc=False),
    scratch_types=[
        pltpu.VMEM((BCH + 1, CH), jnp.int32),
        pltpu.VMEM((BCH + 1, CH), jnp.int32),
        pltpu.VMEM((CH, F), jnp.float32),
        pltpu.VMEM((CH, F), jnp.float32),
        pltpu.VMEM((CH, F), jnp.float32),
        pltpu.VMEM((CH, F), jnp.float32),
        pltpu.VMEM((CH, F), jnp.float32),
        pltpu.VMEM((CH, F), jnp.float32),
        pltpu.VMEM_SHARED((N, F), jnp.float32),
        pltpu.SemaphoreType.DMA,
        pltpu.SemaphoreType.DMA,
        pltpu.SemaphoreType.DMA,
        pltpu.SemaphoreType.DMA,
        pltpu.SemaphoreType.DMA,
        pltpu.SemaphoreType.DMA,
        pltpu.SemaphoreType.DMA,
        pltpu.SemaphoreType.DMA,
        pltpu.SemaphoreType.DMA,
    ],
)(_sc_body)

_final_call = pl.pallas_call(
    _final_body,
    grid=(10,),
    in_specs=[pl.BlockSpec((N // 10, D), lambda i: (i, 0)),
              pl.BlockSpec((NC, N // 10, F), lambda i: (0, i, 0)),
              pl.BlockSpec((D, ON), lambda i: (0, 0)),
              pl.BlockSpec((F, ON), lambda i: (0, 0)),
              pl.BlockSpec((F, ON), lambda i: (0, 0)),
              pl.BlockSpec((1, ON), lambda i: (0, 0)),
              pl.BlockSpec((1, DG + A), lambda i: (0, 0)),
              pl.BlockSpec((DG + A, OG), lambda i: (0, 0)),
              pl.BlockSpec((1, OG), lambda i: (0, 0)),
              pl.BlockSpec((ON, A), lambda i: (0, 0)),
              pl.BlockSpec((OG, A), lambda i: (0, 0)),
              pl.BlockSpec((1, A), lambda i: (0, 0)),
              pl.BlockSpec((1, A), lambda i: (0, 0))],
    out_specs=pl.BlockSpec((1, A), lambda i: (0, 0)),
    out_shape=jax.ShapeDtypeStruct((1, A), jnp.float32),
    scratch_shapes=[pltpu.VMEM((1, ON), jnp.float32)],
)


def kernel(node_features, edge_features, edges_src, edges_dst,
           graph_features, action_mask,
           W_msg, b_msg, W_node, b_node, W_graph, b_graph,
           W_logit, b_logit):
    x = node_features[0]
    ef = edge_features[0]
    src3 = edges_src[0].reshape(TCH, CH)
    dst3 = edges_dst[0].reshape(TCH, CH)

    w1p = W_msg[:D].reshape(D, NC, F).transpose(1, 0, 2)

    xm = _xm_call(x, w1p)
    em = _em_call(ef, W_msg[D:], b_msg.reshape(1, ON))

    zeros = jnp.zeros((RPT, F), dtype=jnp.float32)
    halves = _sc_call(xm, em, src3, dst3, zeros)

    logits = _final_call(
        x, halves, W_node[:D], W_node[D:D + F], W_node[D + F:],
        b_node.reshape(1, ON),
        graph_features, W_graph, b_graph.reshape(1, OG),
        W_logit[:ON], W_logit[ON:], b_logit.reshape(1, A), action_mask)
    return logits


# speedup vs baseline: 1.9968x; 1.0078x over previous
import functools

import jax
import jax.numpy as jnp
from jax import lax
from jax.experimental import pallas as pl
from jax.experimental.pallas import tpu as pltpu
from jax.experimental.pallas import tpu_sc as plsc

N = 10000
E = 320000
D = 128
DE = 16
ON = 128
DG = 64
OG = 64
A = 16

NC = 2
F = ON // NC
NS = 16
CH = 128
TCH = E // CH
BCH = TCH // NS
XCH = TCH - NS * BCH
RPT = 624
TAIL = N - NS * RPT

LANES = 16


def _xm_body(x_ref, w_ref, o_ref):
    o_ref[0] = jnp.dot(x_ref[...], w_ref[0],
                       preferred_element_type=jnp.float32)


def _em_body(ef_ref, w_ref, b_ref, o_ref):
    o_ref[...] = jnp.dot(ef_ref[...], w_ref[...],
                         preferred_element_type=jnp.float32) + b_ref[...]


def _sc_body(xm_hbm, em_hbm, src_hbm, dst_hbm, zero_hbm, out_hbm,
             src_v, dst_v, rows0, rows1, rows2, msg0, msg1, msg2, acc_sh,
             gsem0, gsem1, gsem2, esem0, esem1, esem2,
             ssem0, ssem1, ssem2):
    cid = lax.axis_index("c")
    sid = lax.axis_index("s")
    row0 = sid * RPT
    cbase = sid * BCH
    coff = cid * F
    xm_c = xm_hbm.at[cid]
    slots = ((rows0, msg0, None, gsem0, esem0, ssem0),
             (rows1, msg1, None, gsem1, esem1, ssem1),
             (rows2, msg2, None, gsem2, esem2, ssem2))

    pltpu.sync_copy(src_hbm.at[pl.ds(cbase, BCH)], src_v.at[pl.ds(0, BCH)])
    pltpu.sync_copy(dst_hbm.at[pl.ds(cbase, BCH)], dst_v.at[pl.ds(0, BCH)])

    @pl.when(sid < XCH)
    def _stage_extra():
        pltpu.sync_copy(src_hbm.at[pl.ds(NS * BCH + sid, 1)],
                        src_v.at[pl.ds(BCH, 1)])
        pltpu.sync_copy(dst_hbm.at[pl.ds(NS * BCH + sid, 1)],
                        dst_v.at[pl.ds(BCH, 1)])

    def issue(j, cg, b):
        rows_b, msg_b, _, gsem_b, esem_b, _ = slots[b]
        pltpu.async_copy(xm_c.at[src_v.at[j]], rows_b, gsem_b)
        pltpu.async_copy(
            em_hbm.at[pl.ds(cg * CH, CH), pl.ds(coff, F)], msg_b, esem_b)

    def wait_loads(j, cg, b):
        rows_b, msg_b, _, gsem_b, esem_b, _ = slots[b]
        pltpu.make_async_copy(xm_c.at[src_v.at[j]], rows_b, gsem_b).wait()
        pltpu.make_async_copy(
            em_hbm.at[pl.ds(cg * CH, CH), pl.ds(coff, F)], msg_b,
            esem_b).wait()

    def compute(b):
        rows_b, msg_b, _, _, _, _ = slots[b]

        def quad(q, c2):
            r = q * 4
            for dr in range(4):
                for c in range(F // LANES):
                    so = pl.ds(c * LANES, LANES)
                    rows_b[r + dr, so] = jnp.maximum(
                        rows_b[r + dr, so] + msg_b[r + dr, so], 0.0)
            return c2

        lax.fori_loop(0, CH // 4, quad, 0)

    def scatter(j, b):
        rows_b, _, _, _, _, ssem_b = slots[b]
        pltpu.async_copy(rows_b, acc_sh.at[dst_v.at[j]], ssem_b, add=True)

    def wait_scatter(j, b):
        rows_b, _, _, _, _, ssem_b = slots[b]
        pltpu.make_async_copy(rows_b, acc_sh.at[dst_v.at[j]],
                              ssem_b).wait()

    pltpu.sync_copy(zero_hbm, acc_sh.at[pl.ds(row0, RPT)])

    @pl.when(sid == NS - 1)
    def _zero_tail():
        pltpu.sync_copy(zero_hbm.at[pl.ds(0, TAIL)],
                        acc_sh.at[pl.ds(N - TAIL, TAIL)])

    plsc.subcore_barrier()

    issue(0, cbase + 0, 0)
    issue(1, cbase + 1, 1)
    issue(2, cbase + 2, 2)

    def trip(it, carry):
        j0 = it * 3
        for b in range(3):
            j = j0 + b
            wait_loads(j, cbase + j, b)

            @pl.when(it > 0)
            def _drain():
                wait_scatter(j - 3, b)

            compute(b)
            scatter(j, b)

            @pl.when(j + 3 < BCH)
            def _next():
                issue(j + 3, cbase + j + 3, b)
        return carry

    lax.fori_loop(0, BCH // 3, trip, 0)
    wait_scatter(BCH - 3, 0)
    wait_scatter(BCH - 2, 1)
    wait_scatter(BCH - 1, 2)

    @pl.when(sid < XCH)
    def _extra():
        issue(BCH, NS * BCH + sid, 0)
        wait_loads(BCH, NS * BCH + sid, 0)
        compute(0)
        scatter(BCH, 0)
        wait_scatter(BCH, 0)

    plsc.subcore_barrier()
    pltpu.sync_copy(acc_sh.at[pl.ds(row0, RPT)],
                    out_hbm.at[cid, pl.ds(row0, RPT)])

    @pl.when(sid == NS - 1)
    def _write_tail():
        pltpu.sync_copy(acc_sh.at[pl.ds(N - TAIL, TAIL)],
                        out_hbm.at[cid, pl.ds(N - TAIL, TAIL)])


def _final_body(x_ref, p_ref, wn1_ref, wn2a_ref, wn2b_ref, bn_ref, gf_ref,
                wg_ref, bg_ref, wl1_ref, wl2_ref, bl_ref, am_ref, o_ref,
                acc):
    i = pl.program_id(0)

    @pl.when(i == 0)
    def _init():
        acc[...] = jnp.zeros_like(acc)

    h = jnp.maximum(
        jnp.dot(x_ref[...], wn1_ref[...], preferred_element_type=jnp.float32)
        + jnp.dot(p_ref[0], wn2a_ref[...], preferred_element_type=jnp.float32)
        + jnp.dot(p_ref[1], wn2b_ref[...], preferred_element_type=jnp.float32)
        + bn_ref[...], 0.0)
    acc[...] += jnp.sum(h, axis=0, keepdims=True)

    @pl.when(i == pl.num_programs(0) - 1)
    def _fin():
        emb_nodes = acc[...] / N
        emb_graph = jnp.dot(gf_ref[...], wg_ref[...],
                            preferred_element_type=jnp.float32) + bg_ref[...]
        logits = (jnp.dot(emb_nodes, wl1_ref[...],
                          preferred_element_type=jnp.float32)
                  + jnp.dot(emb_graph, wl2_ref[...],
                            preferred_element_type=jnp.float32)
                  + bl_ref[...])
        inf_mask = jnp.maximum(jnp.log(am_ref[...]),
                               jnp.finfo(jnp.float32).min)
        o_ref[...] = logits + inf_mask


_xm_call = pl.pallas_call(
    _xm_body,
    grid=(NC,),
    in_specs=[pl.BlockSpec((N, D), lambda j: (0, 0)),
              pl.BlockSpec((1, D, F), lambda j: (j, 0, 0))],
    out_specs=pl.BlockSpec((1, N, F), lambda j: (j, 0, 0)),
    out_shape=jax.ShapeDtypeStruct((NC, N, F), jnp.float32),
)

_em_call = pl.pallas_call(
    _em_body,
    grid=(16,),
    in_specs=[pl.BlockSpec((E // 16, DE), lambda i: (i, 0)),
              pl.BlockSpec((DE, ON), lambda i: (0, 0)),
              pl.BlockSpec((1, ON), lambda i: (0, 0))],
    out_specs=pl.BlockSpec((E // 16, ON), lambda i: (i, 0)),
    out_shape=jax.ShapeDtypeStruct((E, ON), jnp.float32),
)

_sc_call = functools.partial(
    pl.kernel,
    out_type=jax.ShapeDtypeStruct((NC, N, F), jnp.float32),
    mesh=plsc.VectorSubcoreMesh(core_axis_name="c", subcore_axis_name="s"),
    compiler_params=pltpu.CompilerParams(use_tc_tiling_on_sc=False),
    scratch_types=[
        pltpu.VMEM((BCH + 1, CH), jnp.int32),
        pltpu.VMEM((BCH + 1, CH), jnp.int32),
        pltpu.VMEM((CH, F), jnp.float32),
        pltpu.VMEM((CH, F), jnp.float32),
        pltpu.VMEM((CH, F), jnp.float32),
        pltpu.VMEM((CH, F), jnp.float32),
        pltpu.VMEM((CH, F), jnp.float32),
        pltpu.VMEM((CH, F), jnp.float32),
        pltpu.VMEM_SHARED((N, F), jnp.float32),
        pltpu.SemaphoreType.DMA,
        pltpu.SemaphoreType.DMA,
        pltpu.SemaphoreType.DMA,
        pltpu.SemaphoreType.DMA,
        pltpu.SemaphoreType.DMA,
        pltpu.SemaphoreType.DMA,
        pltpu.SemaphoreType.DMA,
        pltpu.SemaphoreType.DMA,
        pltpu.SemaphoreType.DMA,
    ],
)(_sc_body)

_final_call = pl.pallas_call(
    _final_body,
    grid=(10,),
    in_specs=[pl.BlockSpec((N // 10, D), lambda i: (i, 0)),
              pl.BlockSpec((NC, N // 10, F), lambda i: (0, i, 0)),
              pl.BlockSpec((D, ON), lambda i: (0, 0)),
              pl.BlockSpec((F, ON), lambda i: (0, 0)),
              pl.BlockSpec((F, ON), lambda i: (0, 0)),
              pl.BlockSpec((1, ON), lambda i: (0, 0)),
              pl.BlockSpec((1, DG + A), lambda i: (0, 0)),
              pl.BlockSpec((DG + A, OG), lambda i: (0, 0)),
              pl.BlockSpec((1, OG), lambda i: (0, 0)),
              pl.BlockSpec((ON, A), lambda i: (0, 0)),
              pl.BlockSpec((OG, A), lambda i: (0, 0)),
              pl.BlockSpec((1, A), lambda i: (0, 0)),
              pl.BlockSpec((1, A), lambda i: (0, 0))],
    out_specs=pl.BlockSpec((1, A), lambda i: (0, 0)),
    out_shape=jax.ShapeDtypeStruct((1, A), jnp.float32),
    scratch_shapes=[pltpu.VMEM((1, ON), jnp.float32)],
)


def kernel(node_features, edge_features, edges_src, edges_dst,
           graph_features, action_mask,
           W_msg, b_msg, W_node, b_node, W_graph, b_graph,
           W_logit, b_logit):
    x = node_features[0]
    ef = edge_features[0]
    src3 = edges_src[0].reshape(TCH, CH)
    dst3 = edges_dst[0].reshape(TCH, CH)

    w1p = W_msg[:D].reshape(D, NC, F).transpose(1, 0, 2)

    xm = _xm_call(x, w1p)
    em = _em_call(ef, W_msg[D:], b_msg.reshape(1, ON))

    zeros = jnp.zeros((RPT, F), dtype=jnp.float32)
    halves = _sc_call(xm, em, src3, dst3, zeros)

    logits = _final_call(
        x, halves, W_node[:D], W_node[D:D + F], W_node[D + F:],
        b_node.reshape(1, ON),
        graph_features, W_graph, b_graph.reshape(1, OG),
        W_logit[:ON], W_logit[ON:], b_logit.reshape(1, A), action_mask)
    return logits
